# Initial kernel scaffold; baseline (speedup 1.0000x reference)
import jax
import jax.numpy as jnp
from jax.experimental import pallas as pl


def kernel(candidate_pts, src_keypts, tgt_pts_xyz, tgt_deep_feat_pts):
    raise NotImplementedError("write your pallas kernel here")


# speedup vs baseline: 5.0277x; 5.0277x over previous
import jax
import jax.numpy as jnp
from jax.experimental import pallas as pl


def _square_distance(src, dst):
    dist = -2.0 * jnp.matmul(src, jnp.swapaxes(dst, 1, 2))
    dist = dist + jnp.sum(src ** 2, axis=-1)[:, :, None]
    dist = dist + jnp.sum(dst ** 2, axis=-1)[:, None, :]
    return dist


def _index_points(points, idx):
    B = points.shape[0]
    batch_idx = jnp.arange(B).reshape((B,) + (1,) * (idx.ndim - 1))
    return points[batch_idx, idx]


from jax.experimental.pallas import tpu as pltpu

_FPS_R, _FPS_C = 8, 256


def _fps_body(xs_ref, ys_ref, zs_ref, x_ref, y_ref, z_ref,
              idx_ref, nx_ref, ny_ref, nz_ref):
    N = _FPS_R * _FPS_C
    x = x_ref[...]
    y = y_ref[...]
    z = z_ref[...]
    iota = (jax.lax.broadcasted_iota(jnp.int32, (_FPS_R, _FPS_C), 0) * _FPS_C
            + jax.lax.broadcasted_iota(jnp.int32, (_FPS_R, _FPS_C), 1))

    def body(i, carry):
        dist, far = carry
        idx_ref[i] = far
        cx = xs_ref[far]
        cy = ys_ref[far]
        cz = zs_ref[far]
        nx_ref[i] = cx
        ny_ref[i] = cy
        nz_ref[i] = cz
        dx = x - cx
        dy = y - cy
        dz = z - cz
        d = (dx * dx + dy * dy) + dz * dz
        dist = jnp.minimum(dist, d)
        m = jnp.max(dist)
        far2 = jnp.min(jnp.where(dist == m, iota, N)).astype(jnp.int32)
        return dist, far2

    init = (jnp.full((_FPS_R, _FPS_C), 1e10, dtype=jnp.float32),
            jnp.array(0, dtype=jnp.int32))
    jax.lax.fori_loop(0, N, body, init)


def _fps_pallas(cand):
    N = cand.shape[0]
    x = cand[:, 0].reshape(_FPS_R, _FPS_C)
    y = cand[:, 1].reshape(_FPS_R, _FPS_C)
    z = cand[:, 2].reshape(_FPS_R, _FPS_C)
    xf, yf, zf = cand[:, 0], cand[:, 1], cand[:, 2]
    smem = pl.BlockSpec(memory_space=pltpu.SMEM)
    out = pl.pallas_call(
        _fps_body,
        in_specs=[smem, smem, smem,
                  pl.BlockSpec(memory_space=pltpu.VMEM),
                  pl.BlockSpec(memory_space=pltpu.VMEM),
                  pl.BlockSpec(memory_space=pltpu.VMEM)],
        out_specs=[smem, smem, smem, smem],
        out_shape=[jax.ShapeDtypeStruct((N,), jnp.int32),
                   jax.ShapeDtypeStruct((N,), jnp.float32),
                   jax.ShapeDtypeStruct((N,), jnp.float32),
                   jax.ShapeDtypeStruct((N,), jnp.float32)],
    )(xf, yf, zf, x, y, z)
    fps_idx, nx, ny, nz = out
    return fps_idx, jnp.stack([nx, ny, nz], axis=-1)


_BQ_BLK = 256


def _bq_body(nq_ref, pT_ref, out_ref):
    N = pT_ref.shape[1]
    nq = nq_ref[...]
    pT = pT_ref[...]
    px = pT[0:1, :]
    py = pT[1:2, :]
    pz = pT[2:3, :]
    ssq_p = (px * px + py * py) + pz * pz
    qx = nq[:, 0:1]
    qy = nq[:, 1:2]
    qz = nq[:, 2:3]
    ssq_q = (qx * qx + qy * qy) + qz * qz
    mm = jnp.dot(nq, pT, preferred_element_type=jnp.float32)
    d = -2.0 * mm
    d = d + ssq_q
    d = d + ssq_p
    iota_p = jax.lax.broadcasted_iota(jnp.int32, d.shape, 1)
    v = jnp.where(d > 1.0, N, iota_p)

    cols = []
    for _ in range(32):
        m = jnp.min(v, axis=1, keepdims=True)
        cols.append(m)
        v = jnp.where(v == m, N, v)
    sel = jnp.concatenate(cols, axis=1)
    sel = jnp.where(sel == N, cols[0], sel)
    out_ref[...] = sel


def _ball_query_pallas(new_xyz, candT):
    N = new_xyz.shape[0]
    return pl.pallas_call(
        _bq_body,
        grid=(N // _BQ_BLK,),
        in_specs=[pl.BlockSpec((_BQ_BLK, 3), lambda i: (i, 0)),
                  pl.BlockSpec((3, N), lambda i: (0, 0))],
        out_specs=pl.BlockSpec((_BQ_BLK, 32), lambda i: (i, 0)),
        out_shape=jax.ShapeDtypeStruct((N, 32), jnp.int32),
    )(new_xyz, candT)


_KNN_BLK = 128


def _knn_body(q_ref, tT_ref, idx_ref, w_ref):
    N = tT_ref.shape[1]
    q = q_ref[...]
    tT = tT_ref[...]
    tx = tT[0:1, :]
    ty = tT[1:2, :]
    tz = tT[2:3, :]
    ssq_t = (tx * tx + ty * ty) + tz * tz
    qx = q[:, 0:1]
    qy = q[:, 1:2]
    qz = q[:, 2:3]
    ssq_q = (qx * qx + qy * qy) + qz * qz
    mm = jnp.dot(q, tT, preferred_element_type=jnp.float32)
    d = -2.0 * mm
    d = d + ssq_q
    d = d + ssq_t
    iota = jax.lax.broadcasted_iota(jnp.int32, d.shape, 1)
    icols = []
    dcols = []
    for _ in range(32):
        m = jnp.min(d, axis=1, keepdims=True)
        ji = jnp.min(jnp.where(d == m, iota, N), axis=1, keepdims=True)
        icols.append(ji)
        dcols.append(jnp.sqrt(jnp.maximum(m, 0.0)))
        d = jnp.where((d == m) & (iota == ji), jnp.inf, d)
    idx_ref[...] = jnp.concatenate(icols, axis=1)
    dist = jnp.concatenate(dcols, axis=1)
    w_ref[...] = dist / jnp.sum(dist, axis=1, keepdims=True)


def _knn_pallas(cand, tgtT):
    Q = cand.shape[0]
    N = tgtT.shape[1]
    return pl.pallas_call(
        _knn_body,
        grid=(Q // _KNN_BLK,),
        in_specs=[pl.BlockSpec((_KNN_BLK, 3), lambda i: (i, 0)),
                  pl.BlockSpec((3, N), lambda i: (0, 0))],
        out_specs=[pl.BlockSpec((_KNN_BLK, 32), lambda i: (i, 0)),
                   pl.BlockSpec((_KNN_BLK, 32), lambda i: (i, 0))],
        out_shape=[jax.ShapeDtypeStruct((Q, 32), jnp.int32),
                   jax.ShapeDtypeStruct((Q, 32), jnp.float32)],
    )(cand, tgtT)


def kernel(candidate_pts, src_keypts, tgt_pts_xyz, tgt_deep_feat_pts):
    B = src_keypts.shape[0]
    cs = candidate_pts.shape
    cand = candidate_pts.reshape(B, cs[1] * cs[2], cs[3])

    fps_idx, new_xyz = _fps_pallas(cand[0])
    fps_idx = fps_idx[None]
    new_xyz = new_xyz[None]
    bq_idx = _ball_query_pallas(new_xyz[0], cand[0].T)[None]
    grouped_xyz = _index_points(cand, bq_idx)
    grouped_xyz_norm = grouped_xyz - new_xyz[:, :, None, :]
    candidate_pts_grouped_xyz = grouped_xyz_norm.reshape(B, cs[1], cs[2], 32, cs[3])

    k_nn = 32
    idx, w = _knn_pallas(cand[0], tgt_pts_xyz[0].T)
    idx = idx[None]
    w = w[None]

    C_deep_feat = tgt_deep_feat_pts.shape[2]
    feats = tgt_deep_feat_pts[0, idx.reshape(-1), :].reshape(
        B, cs[1], cs[2], k_nn, C_deep_feat
    )
    wmap = jnp.broadcast_to(
        w.reshape(B, cs[1], cs[2], k_nn, 1), (B, cs[1], cs[2], k_nn, C_deep_feat)
    )
    return jnp.concatenate([candidate_pts_grouped_xyz, feats * wmap], axis=4)


# speedup vs baseline: 7.8112x; 1.5536x over previous
import jax
import jax.numpy as jnp
from jax.experimental import pallas as pl


def _square_distance(src, dst):
    dist = -2.0 * jnp.matmul(src, jnp.swapaxes(dst, 1, 2))
    dist = dist + jnp.sum(src ** 2, axis=-1)[:, :, None]
    dist = dist + jnp.sum(dst ** 2, axis=-1)[:, None, :]
    return dist


def _index_points(points, idx):
    B = points.shape[0]
    batch_idx = jnp.arange(B).reshape((B,) + (1,) * (idx.ndim - 1))
    return points[batch_idx, idx]


from jax.experimental.pallas import tpu as pltpu

_FPS_R, _FPS_C = 8, 256


def _fps_body(xs_ref, ys_ref, zs_ref, x_ref, y_ref, z_ref,
              idx_ref, nx_ref, ny_ref, nz_ref):
    N = _FPS_R * _FPS_C
    x = x_ref[...]
    y = y_ref[...]
    z = z_ref[...]
    iota = (jax.lax.broadcasted_iota(jnp.int32, (_FPS_R, _FPS_C), 0) * _FPS_C
            + jax.lax.broadcasted_iota(jnp.int32, (_FPS_R, _FPS_C), 1))

    def body(i, carry):
        dist, far = carry
        idx_ref[i] = far
        cx = xs_ref[far]
        cy = ys_ref[far]
        cz = zs_ref[far]
        nx_ref[i] = cx
        ny_ref[i] = cy
        nz_ref[i] = cz
        dx = x - cx
        dy = y - cy
        dz = z - cz
        d = (dx * dx + dy * dy) + dz * dz
        dist = jnp.minimum(dist, d)
        m = jnp.max(dist)
        far2 = jnp.min(jnp.where(dist == m, iota, N)).astype(jnp.int32)
        return dist, far2

    init = (jnp.full((_FPS_R, _FPS_C), 1e10, dtype=jnp.float32),
            jnp.array(0, dtype=jnp.int32))
    jax.lax.fori_loop(0, N, body, init)


def _fps_pallas(cand):
    N = cand.shape[0]
    x = cand[:, 0].reshape(_FPS_R, _FPS_C)
    y = cand[:, 1].reshape(_FPS_R, _FPS_C)
    z = cand[:, 2].reshape(_FPS_R, _FPS_C)
    xf, yf, zf = cand[:, 0], cand[:, 1], cand[:, 2]
    smem = pl.BlockSpec(memory_space=pltpu.SMEM)
    out = pl.pallas_call(
        _fps_body,
        in_specs=[smem, smem, smem,
                  pl.BlockSpec(memory_space=pltpu.VMEM),
                  pl.BlockSpec(memory_space=pltpu.VMEM),
                  pl.BlockSpec(memory_space=pltpu.VMEM)],
        out_specs=[smem, smem, smem, smem],
        out_shape=[jax.ShapeDtypeStruct((N,), jnp.int32),
                   jax.ShapeDtypeStruct((N,), jnp.float32),
                   jax.ShapeDtypeStruct((N,), jnp.float32),
                   jax.ShapeDtypeStruct((N,), jnp.float32)],
    )(xf, yf, zf, x, y, z)
    fps_idx, nx, ny, nz = out
    return fps_idx, nx, ny, nz


_BQ_BLK = 256


def _bq_body(nq_ref, pT_ref, out_ref):
    N = pT_ref.shape[1]
    nq = nq_ref[...]
    pT = pT_ref[...]
    px = pT[0:1, :]
    py = pT[1:2, :]
    pz = pT[2:3, :]
    ssq_p = (px * px + py * py) + pz * pz
    qx = nq[:, 0:1]
    qy = nq[:, 1:2]
    qz = nq[:, 2:3]
    ssq_q = (qx * qx + qy * qy) + qz * qz
    mm = jnp.dot(nq, pT, preferred_element_type=jnp.float32)
    d = -2.0 * mm
    d = d + ssq_q
    d = d + ssq_p
    iota_p = jax.lax.broadcasted_iota(jnp.int32, d.shape, 1)
    v = jnp.where(d > 1.0, N, iota_p)

    cols = []
    for _ in range(32):
        m = jnp.min(v, axis=1, keepdims=True)
        cols.append(m)
        v = jnp.where(v == m, N, v)
    sel = jnp.concatenate(cols, axis=1)
    sel = jnp.where(sel == N, cols[0], sel)
    out_ref[...] = sel


def _ball_query_pallas(new_xyz, candT):
    N = new_xyz.shape[0]
    return pl.pallas_call(
        _bq_body,
        grid=(N // _BQ_BLK,),
        in_specs=[pl.BlockSpec((_BQ_BLK, 3), lambda i: (i, 0)),
                  pl.BlockSpec((3, N), lambda i: (0, 0))],
        out_specs=pl.BlockSpec((_BQ_BLK, 32), lambda i: (i, 0)),
        out_shape=jax.ShapeDtypeStruct((N, 32), jnp.int32),
    )(new_xyz, candT)


_KNN_BLK = 128


def _knn_body(q_ref, tT_ref, idx_ref, w_ref):
    N = tT_ref.shape[1]
    q = q_ref[...]
    tT = tT_ref[...]
    tx = tT[0:1, :]
    ty = tT[1:2, :]
    tz = tT[2:3, :]
    ssq_t = (tx * tx + ty * ty) + tz * tz
    qx = q[:, 0:1]
    qy = q[:, 1:2]
    qz = q[:, 2:3]
    ssq_q = (qx * qx + qy * qy) + qz * qz
    mm = jnp.dot(q, tT, preferred_element_type=jnp.float32)
    d = -2.0 * mm
    d = d + ssq_q
    d = d + ssq_t
    iota = jax.lax.broadcasted_iota(jnp.int32, d.shape, 1)
    icols = []
    dcols = []
    for _ in range(32):
        m = jnp.min(d, axis=1, keepdims=True)
        ji = jnp.min(jnp.where(d == m, iota, N), axis=1, keepdims=True)
        icols.append(ji)
        dcols.append(jnp.sqrt(jnp.maximum(m, 0.0)))
        d = jnp.where((d == m) & (iota == ji), jnp.inf, d)
    idx_ref[...] = jnp.concatenate(icols, axis=1)
    dist = jnp.concatenate(dcols, axis=1)
    w_ref[...] = dist / jnp.sum(dist, axis=1, keepdims=True)


def _knn_pallas(cand, tgtT):
    Q = cand.shape[0]
    N = tgtT.shape[1]
    return pl.pallas_call(
        _knn_body,
        grid=(Q // _KNN_BLK,),
        in_specs=[pl.BlockSpec((_KNN_BLK, 3), lambda i: (i, 0)),
                  pl.BlockSpec((3, N), lambda i: (0, 0))],
        out_specs=[pl.BlockSpec((_KNN_BLK, 32), lambda i: (i, 0)),
                   pl.BlockSpec((_KNN_BLK, 32), lambda i: (i, 0))],
        out_shape=[jax.ShapeDtypeStruct((Q, 32), jnp.int32),
                   jax.ShapeDtypeStruct((Q, 32), jnp.float32)],
    )(cand, tgtT)


import functools
from jax import lax
from jax.experimental.pallas import tpu_sc as plsc

_SC_NC, _SC_NS, _SC_L = 2, 16, 16
_SC_NW = _SC_NC * _SC_NS
_ROWS = 65536
_RPT = _ROWS // _SC_NW


def _sc_gather_body(feat_tbl, knn_flat, bq_flat, qp_flat, ilv_flat,
                    w_flat, cxh, cyh, czh, nxh, nyh, nzh,
                    out_feats, out_gxyzw, scr,
                    kidx_v, bq_v, qp_v, ilv_v,
                    feat_buf, w_v, gx_v, gy_v, gz_v, nx_v, ny_v, nz_v,
                    ilvbuf, semf, semx):
    wid = lax.axis_index("s") * _SC_NC + lax.axis_index("c")
    base = wid * _RPT
    pltpu.sync_copy(knn_flat.at[pl.ds(base, _RPT)], kidx_v)
    featc = pltpu.async_copy(feat_tbl.at[kidx_v], feat_buf, semf)
    pltpu.sync_copy(bq_flat.at[pl.ds(base, _RPT)], bq_v)
    pltpu.sync_copy(qp_flat.at[pl.ds(base, _RPT)], qp_v)
    copies = [
        pltpu.async_copy(cxh.at[bq_v], gx_v, semx),
        pltpu.async_copy(cyh.at[bq_v], gy_v, semx),
        pltpu.async_copy(czh.at[bq_v], gz_v, semx),
        pltpu.async_copy(nxh.at[qp_v], nx_v, semx),
        pltpu.async_copy(nyh.at[qp_v], ny_v, semx),
        pltpu.async_copy(nzh.at[qp_v], nz_v, semx),
    ]
    pltpu.sync_copy(w_flat.at[pl.ds(base, _RPT)], w_v)
    pltpu.sync_copy(ilv_flat.at[pl.ds(base * 4, _RPT * 4)], ilv_v)
    for c in copies:
        c.wait()

    def g_body(g, carry):
        s = pl.ds(g * _SC_L, _SC_L)
        gx_v[s] = gx_v[s] - nx_v[s]
        gy_v[s] = gy_v[s] - ny_v[s]
        gz_v[s] = gz_v[s] - nz_v[s]
        return carry

    lax.fori_loop(0, _RPT // _SC_L, g_body, 0)

    pltpu.sync_copy(gx_v, scr.at[pl.ds(base, _RPT)])
    pltpu.sync_copy(gy_v, scr.at[pl.ds(_ROWS + base, _RPT)])
    pltpu.sync_copy(gz_v, scr.at[pl.ds(2 * _ROWS + base, _RPT)])
    pltpu.sync_copy(w_v, scr.at[pl.ds(3 * _ROWS + base, _RPT)])
    pltpu.async_copy(scr.at[ilv_v], ilvbuf, semx).wait()
    pltpu.sync_copy(ilvbuf, out_gxyzw.at[pl.ds(base * 4, _RPT * 4)])
    featc.wait()
    pltpu.sync_copy(feat_buf, out_feats.at[pl.ds(base, _RPT)])


def _sc_gather(feat_tbl, knn_idx, bq_idx, w, candx, candy, candz, nx, ny, nz):
    mesh = plsc.VectorSubcoreMesh(core_axis_name="c", subcore_axis_name="s")
    knn_flat = knn_idx.reshape(-1)
    bq_flat = bq_idx.reshape(-1)
    rows = jnp.arange(_ROWS, dtype=jnp.int32)
    qp_flat = rows // 32
    irows = jnp.arange(_ROWS * 4, dtype=jnp.int32)
    ilv_flat = (irows % 4) * _ROWS + irows // 4
    w_flat = w.reshape(-1)
    iplane = pltpu.VMEM((_RPT,), jnp.int32)
    plane = pltpu.VMEM((_RPT,), jnp.float32)
    run = functools.partial(
        pl.kernel, mesh=mesh,
        compiler_params=pltpu.CompilerParams(use_tc_tiling_on_sc=False),
        out_type=[jax.ShapeDtypeStruct((_ROWS, 32), jnp.float32),
                  jax.ShapeDtypeStruct((_ROWS * 4,), jnp.float32),
                  jax.ShapeDtypeStruct((_ROWS * 4,), jnp.float32)],
        scratch_types=[
            iplane, iplane, iplane,
            pltpu.VMEM((_RPT * 4,), jnp.int32),
            pltpu.VMEM((_RPT, 32), jnp.float32),
            plane, plane, plane, plane, plane, plane, plane,
            pltpu.VMEM((_RPT * 4,), jnp.float32),
            pltpu.SemaphoreType.DMA,
            pltpu.SemaphoreType.DMA,
        ],
    )(_sc_gather_body)
    feats, gxyzw, _ = run(feat_tbl, knn_flat, bq_flat, qp_flat, ilv_flat,
                          w_flat, candx, candy, candz, nx, ny, nz)
    return feats, gxyzw.reshape(_ROWS, 4)


_ASM_BLK = 2048


def _asm_body(g_ref, f_ref, o_ref):
    g = g_ref[...]
    f = f_ref[...]
    o_ref[...] = jnp.concatenate([g[:, 0:3], f * g[:, 3:4]], axis=1)


def _assembly_pallas(gxyzw, feats):
    R = gxyzw.shape[0]
    return pl.pallas_call(
        _asm_body,
        grid=(R // _ASM_BLK,),
        in_specs=[pl.BlockSpec((_ASM_BLK, 4), lambda i: (i, 0)),
                  pl.BlockSpec((_ASM_BLK, 32), lambda i: (i, 0))],
        out_specs=pl.BlockSpec((_ASM_BLK, 35), lambda i: (i, 0)),
        out_shape=jax.ShapeDtypeStruct((R, 35), jnp.float32),
    )(gxyzw, feats)


def kernel(candidate_pts, src_keypts, tgt_pts_xyz, tgt_deep_feat_pts):
    B = src_keypts.shape[0]
    cs = candidate_pts.shape
    cand = candidate_pts.reshape(B, cs[1] * cs[2], cs[3])[0]

    fps_idx, nx, ny, nz = _fps_pallas(cand)
    new_xyz = jnp.stack([nx, ny, nz], axis=-1)
    bq_idx = _ball_query_pallas(new_xyz, cand.T)
    knn_idx, w = _knn_pallas(cand, tgt_pts_xyz[0].T)
    feats, gxyzw = _sc_gather(tgt_deep_feat_pts[0], knn_idx, bq_idx, w,
                              cand[:, 0], cand[:, 1], cand[:, 2], nx, ny, nz)
    out = _assembly_pallas(gxyzw, feats)
    return out.reshape(B, cs[1], cs[2], 32, 35)
